# baseline (speedup 1.0000x reference)
import functools
import jax
import jax.numpy as jnp
from jax import lax
from jax.experimental import pallas as pl
from jax.experimental.pallas import tpu as pltpu
from jax.experimental.pallas import tpu_sc as plsc

B = 16384
N = 1000
BLOCK_B = 2048
NW = 32
RPW = B // NW
CHUNK = 32
NCH = RPW // CHUNK


def _reduce_body(x_ref, g_ref, idx_ref, val_ref):
    z = x_ref[...] + g_ref[...]
    m = jnp.max(z, axis=1, keepdims=True)
    ssum = jnp.sum(jnp.exp(z - m), axis=1, keepdims=True)
    s = 1.0 / ssum
    val_ref[...] = (1.0 - s) + s
    cols = lax.broadcasted_iota(jnp.int32, z.shape, 1)
    idx_ref[...] = jnp.min(jnp.where(z == m, cols, N), axis=1, keepdims=True)


_mesh = plsc.VectorSubcoreMesh(core_axis_name="c", subcore_axis_name="s")


@functools.partial(
    pl.kernel, mesh=_mesh,
    compiler_params=pltpu.CompilerParams(needs_layout_passes=False),
    out_type=jax.ShapeDtypeStruct((B, N), jnp.float32),
    scratch_types=[pltpu.VMEM((CHUNK, N), jnp.float32),
                   pltpu.VMEM((CHUNK, N), jnp.float32),
                   pltpu.VMEM((RPW,), jnp.int32),
                   pltpu.VMEM((RPW,), jnp.float32),
                   pltpu.SemaphoreType.DMA,
                   pltpu.SemaphoreType.DMA])
def _sc_scatter(idx_hbm, val_hbm, out_hbm, buf0, buf1, idx_v, val_v, so0, so1):
    c = lax.axis_index("c")
    s = lax.axis_index("s")
    wid = s * 2 + c
    base = wid * RPW

    pltpu.sync_copy(idx_hbm.at[pl.ds(base, RPW)], idx_v)
    pltpu.sync_copy(val_hbm.at[pl.ds(base, RPW)], val_v)

    bufs = (buf0, buf1)
    sout = (so0, so1)
    lanes = lax.iota(jnp.int32, 16)
    zero_row = jnp.zeros((16,), jnp.int32)
    one_col = jnp.full((16,), 1, jnp.int32)
    zeros_f = jnp.zeros((16,), jnp.float32)
    ones_f = jnp.ones((16,), jnp.float32)

    for b in range(2):
        buf = bufs[b]

        def _zbody(r, carry, buf=buf):
            for k in range(N // 16):
                buf[r, pl.ds(k * 16, 16)] = zeros_f
            buf[r, pl.ds(N - 16, 16)] = zeros_f
            return carry

        lax.fori_loop(0, CHUNK, _zbody, 0)

    def _body(i, carry):
        for b in range(2):
            @pl.when(lax.rem(i, 2) == b)
            def _():
                @pl.when(i >= 2)
                def _():
                    pltpu.make_async_copy(
                        bufs[b], out_hbm.at[pl.ds(0, CHUNK), :], sout[b]).wait()
                    for g in range(CHUNK // 16):
                        off = (i - 2) * CHUNK + g * 16
                        rows = g * 16 + lanes
                        cols = idx_v[pl.ds(off, 16)]
                        plsc.store_scatter(bufs[b], [rows, cols], zeros_f)

                    @pl.when((wid == 0) & (i == 2))
                    def _():
                        plsc.store_scatter(bufs[b], [zero_row, one_col],
                                           zeros_f, mask=lanes == 0)

                for g in range(CHUNK // 16):
                    off = i * CHUNK + g * 16
                    rows = g * 16 + lanes
                    cols = idx_v[pl.ds(off, 16)]
                    vals = val_v[pl.ds(off, 16)]
                    plsc.store_scatter(bufs[b], [rows, cols], vals)

                @pl.when((wid == 0) & (i == 0))
                def _():
                    plsc.store_scatter(bufs[b], [zero_row, one_col],
                                       ones_f, mask=lanes == 0)

                pltpu.async_copy(
                    bufs[b], out_hbm.at[pl.ds(base + i * CHUNK, CHUNK), :], sout[b])
        return carry

    lax.fori_loop(0, NCH, _body, 0)
    pltpu.make_async_copy(
        bufs[(NCH - 1) % 2], out_hbm.at[pl.ds(0, CHUNK), :], sout[(NCH - 1) % 2]).wait()
    pltpu.make_async_copy(
        bufs[NCH % 2], out_hbm.at[pl.ds(0, CHUNK), :], sout[NCH % 2]).wait()


def kernel(x, gumbels):
    idx2, val2 = pl.pallas_call(
        _reduce_body,
        grid=(B // BLOCK_B,),
        in_specs=[pl.BlockSpec((BLOCK_B, N), lambda i: (i, 0)),
                  pl.BlockSpec((BLOCK_B, N), lambda i: (i, 0))],
        out_specs=[pl.BlockSpec((BLOCK_B, 1), lambda i: (i, 0)),
                   pl.BlockSpec((BLOCK_B, 1), lambda i: (i, 0))],
        out_shape=[jax.ShapeDtypeStruct((B, 1), jnp.int32),
                   jax.ShapeDtypeStruct((B, 1), jnp.float32)],
        compiler_params=pltpu.CompilerParams(dimension_semantics=("parallel",)),
    )(x, gumbels)
    return _sc_scatter(idx2.reshape(B), val2.reshape(B))


# speedup vs baseline: 1.0049x; 1.0049x over previous
import functools
import jax
import jax.numpy as jnp
from jax import lax
from jax.experimental import pallas as pl
from jax.experimental.pallas import tpu as pltpu
from jax.experimental.pallas import tpu_sc as plsc

B = 16384
N = 1000
BLOCK_B = 2048
R = 10240
NW = 32
RPW = R // NW
CHUNK = 32
NCH = RPW // CHUNK


def _reduce_body(x_ref, g_ref, idx_ref, val_ref):
    z = x_ref[...] + g_ref[...]
    m = jnp.max(z, axis=1, keepdims=True)
    ssum = jnp.sum(jnp.exp(z - m), axis=1, keepdims=True)
    s = 1.0 / ssum
    val_ref[...] = (1.0 - s) + s
    cols = lax.broadcasted_iota(jnp.int32, z.shape, 1)
    idx_ref[...] = jnp.min(jnp.where(z == m, cols, N), axis=1, keepdims=True)


def _reduce(x, gumbels, row0, nrows):
    blk0 = row0 // BLOCK_B
    return pl.pallas_call(
        _reduce_body,
        grid=(nrows // BLOCK_B,),
        in_specs=[pl.BlockSpec((BLOCK_B, N), lambda i: (i + blk0, 0)),
                  pl.BlockSpec((BLOCK_B, N), lambda i: (i + blk0, 0))],
        out_specs=[pl.BlockSpec((BLOCK_B, 1), lambda i: (i, 0)),
                   pl.BlockSpec((BLOCK_B, 1), lambda i: (i, 0))],
        out_shape=[jax.ShapeDtypeStruct((nrows, 1), jnp.int32),
                   jax.ShapeDtypeStruct((nrows, 1), jnp.float32)],
        compiler_params=pltpu.CompilerParams(dimension_semantics=("parallel",)),
    )(x, gumbels)


_mesh = plsc.VectorSubcoreMesh(core_axis_name="c", subcore_axis_name="s")


@functools.partial(
    pl.kernel, mesh=_mesh,
    compiler_params=pltpu.CompilerParams(needs_layout_passes=False),
    out_type=jax.ShapeDtypeStruct((B, N), jnp.float32),
    scratch_types=[pltpu.VMEM((CHUNK, N), jnp.float32),
                   pltpu.VMEM((CHUNK, N), jnp.float32),
                   pltpu.VMEM((RPW,), jnp.int32),
                   pltpu.VMEM((RPW,), jnp.float32),
                   pltpu.SemaphoreType.DMA,
                   pltpu.SemaphoreType.DMA])
def _sc_scatter(idx_hbm, val_hbm, out_hbm, buf0, buf1, idx_v, val_v, so0, so1):
    c = lax.axis_index("c")
    s = lax.axis_index("s")
    wid = s * 2 + c
    base = wid * RPW

    pltpu.sync_copy(idx_hbm.at[pl.ds(base, RPW)], idx_v)
    pltpu.sync_copy(val_hbm.at[pl.ds(base, RPW)], val_v)

    bufs = (buf0, buf1)
    sout = (so0, so1)
    lanes = lax.iota(jnp.int32, 16)
    zero_row = jnp.zeros((16,), jnp.int32)
    one_col = jnp.full((16,), 1, jnp.int32)
    zeros_f = jnp.zeros((16,), jnp.float32)
    ones_f = jnp.ones((16,), jnp.float32)

    for b in range(2):
        buf = bufs[b]

        def _zbody(r, carry, buf=buf):
            for k in range(N // 16):
                buf[r, pl.ds(k * 16, 16)] = zeros_f
            buf[r, pl.ds(N - 16, 16)] = zeros_f
            return carry

        lax.fori_loop(0, CHUNK, _zbody, 0)

    def _body(i, carry):
        for b in range(2):
            @pl.when(lax.rem(i, 2) == b)
            def _():
                @pl.when(i >= 2)
                def _():
                    pltpu.make_async_copy(
                        bufs[b], out_hbm.at[pl.ds(0, CHUNK), :], sout[b]).wait()
                    for g in range(CHUNK // 16):
                        off = (i - 2) * CHUNK + g * 16
                        rows = g * 16 + lanes
                        cols = idx_v[pl.ds(off, 16)]
                        plsc.store_scatter(bufs[b], [rows, cols], zeros_f)

                    @pl.when((wid == 0) & (i == 2))
                    def _():
                        plsc.store_scatter(bufs[b], [zero_row, one_col],
                                           zeros_f, mask=lanes == 0)

                for g in range(CHUNK // 16):
                    off = i * CHUNK + g * 16
                    rows = g * 16 + lanes
                    cols = idx_v[pl.ds(off, 16)]
                    vals = val_v[pl.ds(off, 16)]
                    plsc.store_scatter(bufs[b], [rows, cols], vals)

                @pl.when((wid == 0) & (i == 0))
                def _():
                    plsc.store_scatter(bufs[b], [zero_row, one_col],
                                       ones_f, mask=lanes == 0)

                pltpu.async_copy(
                    bufs[b], out_hbm.at[pl.ds(base + i * CHUNK, CHUNK), :], sout[b])
        return carry

    lax.fori_loop(0, NCH, _body, 0)
    pltpu.make_async_copy(
        bufs[(NCH - 1) % 2], out_hbm.at[pl.ds(0, CHUNK), :], sout[(NCH - 1) % 2]).wait()
    pltpu.make_async_copy(
        bufs[NCH % 2], out_hbm.at[pl.ds(0, CHUNK), :], sout[NCH % 2]).wait()


def _writer_body(idx_ref, val_ref, alias_ref, out_ref):
    del alias_ref
    cols = lax.broadcasted_iota(jnp.int32, (BLOCK_B, N), 1)
    out_ref[...] = jnp.where(cols == idx_ref[...], val_ref[...], 0.0)


def _write_tail(idx, val, outA):
    nrows = B - R
    blk0 = R // BLOCK_B
    return pl.pallas_call(
        _writer_body,
        grid=(nrows // BLOCK_B,),
        in_specs=[pl.BlockSpec((BLOCK_B, 1), lambda i: (i, 0)),
                  pl.BlockSpec((BLOCK_B, 1), lambda i: (i, 0)),
                  pl.BlockSpec(memory_space=pl.ANY)],
        out_specs=pl.BlockSpec((BLOCK_B, N), lambda i: (i + blk0, 0)),
        out_shape=jax.ShapeDtypeStruct((B, N), jnp.float32),
        input_output_aliases={2: 0},
        compiler_params=pltpu.CompilerParams(dimension_semantics=("arbitrary",)),
    )(idx, val, outA)


def kernel(x, gumbels):
    idx_a, val_a = _reduce(x, gumbels, 0, R)
    outA = _sc_scatter(idx_a.reshape(R), val_a.reshape(R))
    idx_b, val_b = _reduce(x, gumbels, R, B - R)
    return _write_tail(idx_b, val_b, outA)
